# baseline (speedup 1.0000x reference)
import functools

import jax
import jax.numpy as jnp
from jax import lax
from jax.experimental import pallas as pl
from jax.experimental.pallas import tpu as pltpu
from jax.experimental.pallas import tpu_sc as plsc

B, C, H, W = 2, 256, 320, 320
N = 512
NV = B * N
NWORK = 32
VPW = NV // NWORK
RPW = 2 * VPW
NROWS = NV * (C // 128)


def _sc_gather(fm_rows, pos2d):
    mesh = plsc.VectorSubcoreMesh(core_axis_name="c", subcore_axis_name="s")

    @functools.partial(
        pl.kernel,
        out_type=jax.ShapeDtypeStruct((NROWS, 128), jnp.float32),
        mesh=mesh,
        scratch_types=[
            pltpu.VMEM((VPW,), jnp.int32),
            pltpu.VMEM((VPW,), jnp.int32),
            pltpu.VMEM((RPW,), jnp.int32),
            pltpu.VMEM((RPW, 128), jnp.float32),
            pltpu.SemaphoreType.DMA,
        ],
        compiler_params=pltpu.CompilerParams(needs_layout_passes=False),
    )
    def body(fm_hbm, pos_hbm, out_hbm, rows_v, cols_v, idx_v, dat_v, sem):
        wid = lax.axis_index("s") * 2 + lax.axis_index("c")
        v0 = wid * VPW
        bq = lax.shift_right_logical(v0, 9)
        k0 = lax.shift_right_logical(v0 & 511, 7)
        o = v0 & 127
        base = pl.multiple_of((bq * 8 + k0 * 2) * 128 + o, VPW)
        pltpu.sync_copy(pos_hbm.at[pl.ds(base, VPW)], rows_v)
        pltpu.sync_copy(pos_hbm.at[pl.ds(base + 128, VPW)], cols_v)

        lane = jax.lax.iota(jnp.int32, 16)
        for vc in range(VPW // 16):
            v_loc = vc * 16 + lane
            r = rows_v[pl.ds(vc * 16, 16)]
            c = cols_v[pl.ds(vc * 16, 16)]
            b = lax.shift_right_logical(v0 + vc * 16 + lane, 9)
            rho0 = ((b * H + r) * (W // 8) + lax.shift_right_logical(c, 3)) \
                * 16 + (c & 7)
            pos = v_loc * 2
            plsc.store_scatter(idx_v, [pos], rho0)
            plsc.store_scatter(idx_v, [pos + 1], rho0 + 8)

        pltpu.async_copy(fm_hbm.at[idx_v], dat_v, sem).wait()
        pltpu.sync_copy(dat_v, out_hbm.at[pl.ds(wid * RPW, RPW)])

    return body(fm_rows, pos2d)


def kernel(feature_map, vertices_positions):
    fm_rows = (
        feature_map.transpose(0, 2, 3, 1)
        .reshape(B, H, W // 8, 8, C // 128, 128)
        .transpose(0, 1, 2, 4, 3, 5)
        .reshape(B * H * (W // 8) * (C // 128) * 8, 128)
    )
    pos16 = (
        vertices_positions.astype(jnp.int32)
        .transpose(0, 2, 1)
        .reshape(B, 2, N // 128, 128)
        .transpose(0, 2, 1, 3)
        .reshape(B * 2 * N)
    )
    out = _sc_gather(fm_rows, pos16)
    return out.reshape(B, N, C)


# speedup vs baseline: 1.0013x; 1.0013x over previous
import functools

import jax
import jax.numpy as jnp
from jax import lax
from jax.experimental import pallas as pl
from jax.experimental.pallas import tpu as pltpu
from jax.experimental.pallas import tpu_sc as plsc

B, C, H, W = 2, 256, 320, 320
N = 512
NV = B * N
NWORK = 32
VPW = NV // NWORK
RPW = 2 * VPW
NROWS = NV * (C // 128)


def _sc_gather(fm_rows, pos2d):
    mesh = plsc.VectorSubcoreMesh(core_axis_name="c", subcore_axis_name="s")

    @functools.partial(
        pl.kernel,
        out_type=jax.ShapeDtypeStruct((NROWS, 128), jnp.float32),
        mesh=mesh,
        scratch_types=[
            pltpu.VMEM((VPW,), jnp.int32),
            pltpu.VMEM((VPW,), jnp.int32),
            pltpu.VMEM((RPW,), jnp.int32),
            pltpu.VMEM((RPW, 128), jnp.float32),
            pltpu.SemaphoreType.DMA,
        ],
        compiler_params=pltpu.CompilerParams(
            needs_layout_passes=False,
            skip_device_barrier=True,
            disable_bounds_checks=True,
            disable_semaphore_checks=True,
        ),
    )
    def body(fm_hbm, pos_hbm, out_hbm, rows_v, cols_v, idx_v, dat_v, sem):
        wid = lax.axis_index("s") * 2 + lax.axis_index("c")
        v0 = wid * VPW
        bq = lax.shift_right_logical(v0, 9)
        k0 = lax.shift_right_logical(v0 & 511, 7)
        o = v0 & 127
        base = pl.multiple_of((bq * 8 + k0 * 2) * 128 + o, VPW)
        pltpu.sync_copy(pos_hbm.at[pl.ds(base, VPW)], rows_v)
        pltpu.sync_copy(pos_hbm.at[pl.ds(base + 128, VPW)], cols_v)

        lane = jax.lax.iota(jnp.int32, 16)
        for vc in range(VPW // 16):
            v_loc = vc * 16 + lane
            r = rows_v[pl.ds(vc * 16, 16)]
            c = cols_v[pl.ds(vc * 16, 16)]
            b = lax.shift_right_logical(v0 + vc * 16 + lane, 9)
            rho0 = ((b * H + r) * (W // 8) + lax.shift_right_logical(c, 3)) \
                * 16 + (c & 7)
            pos = v_loc * 2
            plsc.store_scatter(idx_v, [pos], rho0)
            plsc.store_scatter(idx_v, [pos + 1], rho0 + 8)

        pltpu.async_copy(fm_hbm.at[idx_v], dat_v, sem).wait()
        pltpu.sync_copy(dat_v, out_hbm.at[pl.ds(wid * RPW, RPW)])

    return body(fm_rows, pos2d)


def kernel(feature_map, vertices_positions):
    fm_rows = (
        feature_map.transpose(0, 2, 3, 1)
        .reshape(B, H, W // 8, 8, C // 128, 128)
        .transpose(0, 1, 2, 4, 3, 5)
        .reshape(B * H * (W // 8) * (C // 128) * 8, 128)
    )
    pos16 = (
        vertices_positions.astype(jnp.int32)
        .transpose(0, 2, 1)
        .reshape(B, 2, N // 128, 128)
        .transpose(0, 2, 1, 3)
        .reshape(B * 2 * N)
    )
    out = _sc_gather(fm_rows, pos16)
    return out.reshape(B, N, C)


# speedup vs baseline: 1.0230x; 1.0216x over previous
import functools

import jax
import jax.numpy as jnp
from jax import lax
from jax.experimental import pallas as pl
from jax.experimental.pallas import tpu as pltpu
from jax.experimental.pallas import tpu_sc as plsc

B, C, H, W = 2, 256, 320, 320
N = 512
NV = B * N
NWORK = 32
VPW = NV // NWORK
RPW = 2 * VPW
NROWS = NV * (C // 128)


def _sc_gather(fm_rows, pos2d):
    mesh = plsc.VectorSubcoreMesh(core_axis_name="c", subcore_axis_name="s")

    @functools.partial(
        pl.kernel,
        out_type=jax.ShapeDtypeStruct((NROWS, 128), jnp.float32),
        mesh=mesh,
        scratch_types=[
            pltpu.VMEM((160,), jnp.int32),
            pltpu.VMEM((RPW,), jnp.int32),
            pltpu.VMEM((RPW, 128), jnp.float32),
            pltpu.SemaphoreType.DMA,
            pltpu.SemaphoreType.DMA,
        ],
        compiler_params=pltpu.CompilerParams(
            needs_layout_passes=False,
            skip_device_barrier=True,
            disable_bounds_checks=True,
            disable_semaphore_checks=True,
        ),
    )
    def body(fm_hbm, pos_hbm, out_hbm, pos_v, idx_v, dat_v, gsem, osem):
        wid = lax.axis_index("s") * 2 + lax.axis_index("c")
        v0 = wid * VPW
        bq = lax.shift_right_logical(v0, 9)
        k0 = lax.shift_right_logical(v0 & 511, 7)
        o = v0 & 127
        base = pl.multiple_of((bq * 8 + k0 * 2) * 128 + o, VPW)
        pltpu.sync_copy(pos_hbm.at[pl.ds(base, 160)], pos_v)

        lane = jax.lax.iota(jnp.int32, 16)
        for vc in range(VPW // 16):
            v_loc = vc * 16 + lane
            r = pos_v[pl.ds(vc * 16, 16)]
            c = pos_v[pl.ds(128 + vc * 16, 16)]
            b = lax.shift_right_logical(v0 + vc * 16 + lane, 9)
            rho0 = ((b * H + r) * (W // 8) + lax.shift_right_logical(c, 3)) \
                * 16 + (c & 7)
            pos = v_loc * 2
            plsc.store_scatter(idx_v, [pos], rho0)
            plsc.store_scatter(idx_v, [pos + 1], rho0 + 8)

        hg = RPW // 2
        g0 = pltpu.async_copy(fm_hbm.at[idx_v.at[pl.ds(0, hg)]],
                              dat_v.at[pl.ds(0, hg)], gsem)
        g1 = pltpu.async_copy(fm_hbm.at[idx_v.at[pl.ds(hg, hg)]],
                              dat_v.at[pl.ds(hg, hg)], gsem)
        g0.wait()
        o0 = pltpu.async_copy(dat_v.at[pl.ds(0, hg)],
                              out_hbm.at[pl.ds(wid * RPW, hg)], osem)
        g1.wait()
        o1 = pltpu.async_copy(dat_v.at[pl.ds(hg, hg)],
                              out_hbm.at[pl.ds(wid * RPW + hg, hg)], osem)
        o0.wait()
        o1.wait()

    return body(fm_rows, pos2d)


def kernel(feature_map, vertices_positions):
    fm_rows = (
        feature_map.transpose(0, 2, 3, 1)
        .reshape(B, H, W // 8, 8, C // 128, 128)
        .transpose(0, 1, 2, 4, 3, 5)
        .reshape(B * H * (W // 8) * (C // 128) * 8, 128)
    )
    pos16 = (
        vertices_positions.astype(jnp.int32)
        .transpose(0, 2, 1)
        .reshape(B, 2, N // 128, 128)
        .transpose(0, 2, 1, 3)
        .reshape(B * 2 * N)
    )
    out = _sc_gather(fm_rows, pos16)
    return out.reshape(B, N, C)


# speedup vs baseline: 1.0261x; 1.0030x over previous
import functools

import jax
import jax.numpy as jnp
from jax import lax
from jax.experimental import pallas as pl
from jax.experimental.pallas import tpu as pltpu
from jax.experimental.pallas import tpu_sc as plsc

B, C, H, W = 2, 256, 320, 320
N = 512
NV = B * N
NWORK = 32
VPW = NV // NWORK
RPW = 2 * VPW
NROWS = NV * (C // 128)


def _sc_gather(fm_rows, pos2d):
    mesh = plsc.VectorSubcoreMesh(core_axis_name="c", subcore_axis_name="s")

    @functools.partial(
        pl.kernel,
        out_type=jax.ShapeDtypeStruct((NROWS, 128), jnp.float32),
        mesh=mesh,
        scratch_types=[
            pltpu.VMEM((160,), jnp.int32),
            pltpu.VMEM((RPW,), jnp.int32),
            pltpu.VMEM((RPW, 128), jnp.float32),
            pltpu.SemaphoreType.DMA,
            pltpu.SemaphoreType.DMA,
            pltpu.SemaphoreType.DMA,
        ],
        compiler_params=pltpu.CompilerParams(
            needs_layout_passes=False,
            skip_device_barrier=True,
            disable_bounds_checks=True,
            disable_semaphore_checks=True,
        ),
    )
    def body(fm_hbm, pos_hbm, out_hbm, pos_v, idx_v, dat_v, gsem0, gsem1,
             osem):
        wid = lax.axis_index("s") * 2 + lax.axis_index("c")
        v0 = wid * VPW
        bq = lax.shift_right_logical(v0, 9)
        k0 = lax.shift_right_logical(v0 & 511, 7)
        o = v0 & 127
        base = pl.multiple_of((bq * 8 + k0 * 2) * 128 + o, VPW)
        pltpu.sync_copy(pos_hbm.at[pl.ds(base, 160)], pos_v)

        lane = jax.lax.iota(jnp.int32, 16)
        hg = RPW // 2
        gathers = []
        for vc, gsem in zip(range(VPW // 16), (gsem0, gsem1)):
            v_loc = vc * 16 + lane
            r = pos_v[pl.ds(vc * 16, 16)]
            c = pos_v[pl.ds(128 + vc * 16, 16)]
            b = lax.shift_right_logical(v0 + vc * 16 + lane, 9)
            rho0 = ((b * H + r) * (W // 8) + lax.shift_right_logical(c, 3)) \
                * 16 + (c & 7)
            pos = v_loc * 2
            plsc.store_scatter(idx_v, [pos], rho0)
            plsc.store_scatter(idx_v, [pos + 1], rho0 + 8)
            gathers.append(
                pltpu.async_copy(fm_hbm.at[idx_v.at[pl.ds(vc * hg, hg)]],
                                 dat_v.at[pl.ds(vc * hg, hg)], gsem))

        outs = []
        for vc in range(VPW // 16):
            gathers[vc].wait()
            outs.append(
                pltpu.async_copy(dat_v.at[pl.ds(vc * hg, hg)],
                                 out_hbm.at[pl.ds(wid * RPW + vc * hg, hg)],
                                 osem))
        for o in outs:
            o.wait()

    return body(fm_rows, pos2d)


def kernel(feature_map, vertices_positions):
    fm_rows = (
        feature_map.transpose(0, 2, 3, 1)
        .reshape(B, H, W // 8, 8, C // 128, 128)
        .transpose(0, 1, 2, 4, 3, 5)
        .reshape(B * H * (W // 8) * (C // 128) * 8, 128)
    )
    pos16 = (
        vertices_positions.astype(jnp.int32)
        .transpose(0, 2, 1)
        .reshape(B, 2, N // 128, 128)
        .transpose(0, 2, 1, 3)
        .reshape(B * 2 * N)
    )
    out = _sc_gather(fm_rows, pos16)
    return out.reshape(B, N, C)
